# Initial kernel scaffold; baseline (speedup 1.0000x reference)
import jax
import jax.numpy as jnp
from jax import lax
from jax.experimental import pallas as pl
from jax.experimental.pallas import tpu as pltpu
from jax.experimental.pallas import tpu_sc as plsc

EMBED_DIM = 32
BATCH = 16384
HIST = 200

NC = 2
NS = 16
NW = NC * NS

TOTAL = BATCH * HIST
PER_W = TOTAL // NW
CHUNK = 2048
NCHUNK = PER_W // CHUNK

assert PER_W * NW == TOTAL and NCHUNK * CHUNK == PER_W


def _body(data_hbm, table_hbm, out_hbm, idx_v, rows_v, sem):
    c = lax.axis_index("c")
    s = lax.axis_index("s")
    wid = s * NC + c
    base = wid * PER_W

    def chunk_step(i, carry):
        off = base + i * CHUNK
        pltpu.sync_copy(data_hbm.at[pl.ds(off, CHUNK)], idx_v)
        pltpu.async_copy(table_hbm.at[idx_v], rows_v, sem).wait()
        pltpu.sync_copy(rows_v, out_hbm.at[pl.ds(off, CHUNK)])
        return carry

    lax.fori_loop(0, NCHUNK, chunk_step, 0)


@jax.jit
def kernel(data, ivectors):
    flat_idx = data.reshape(TOTAL)
    mesh = plsc.VectorSubcoreMesh(core_axis_name="c", subcore_axis_name="s")
    out = pl.kernel(
        _body,
        out_type=jax.ShapeDtypeStruct((TOTAL, EMBED_DIM), jnp.float32),
        mesh=mesh,
        scratch_types=[
            pltpu.VMEM((CHUNK,), jnp.int32),
            pltpu.VMEM((CHUNK, EMBED_DIM), jnp.float32),
            pltpu.SemaphoreType.DMA,
        ],
    )(flat_idx, ivectors)
    return out.reshape(BATCH, HIST, EMBED_DIM)


# speedup vs baseline: 4.9404x; 4.9404x over previous
import jax
import jax.numpy as jnp
from jax import lax
from jax.experimental import pallas as pl
from jax.experimental.pallas import tpu as pltpu
from jax.experimental.pallas import tpu_sc as plsc

EMBED_DIM = 32
BATCH = 16384
HIST = 200

NC = 2
NS = 16
NW = NC * NS

TOTAL = BATCH * HIST
PER_W = TOTAL // NW
CHUNK = 2048
NCHUNK = PER_W // CHUNK

assert PER_W * NW == TOTAL and NCHUNK * CHUNK == PER_W


def _body(data_hbm, table_hbm, out_hbm, idx_v, rows_v, sem):
    c = lax.axis_index("c")
    s = lax.axis_index("s")
    wid = s * NC + c
    base = wid * PER_W

    def chunk_step(i, carry):
        off = base + i * CHUNK
        pltpu.sync_copy(data_hbm.at[pl.ds(off, CHUNK)], idx_v)
        pltpu.async_copy(table_hbm.at[idx_v], rows_v, sem).wait()
        pltpu.sync_copy(rows_v, out_hbm.at[pl.ds(off, CHUNK)])
        return carry

    lax.fori_loop(0, NCHUNK, chunk_step, 0)


@jax.jit
def kernel(data, ivectors):
    flat_idx = data.reshape(TOTAL)
    mesh = plsc.VectorSubcoreMesh(core_axis_name="c", subcore_axis_name="s")
    out = pl.kernel(
        _body,
        out_type=jax.ShapeDtypeStruct((TOTAL, EMBED_DIM), jnp.float32),
        mesh=mesh,
        scratch_types=[
            pltpu.VMEM((CHUNK,), jnp.int32),
            pltpu.VMEM((CHUNK, EMBED_DIM), jnp.float32),
            pltpu.SemaphoreType.DMA,
        ],
        compiler_params=pltpu.CompilerParams(use_tc_tiling_on_sc=False),
    )(flat_idx, ivectors)
    return out.reshape(BATCH, HIST, EMBED_DIM)


# speedup vs baseline: 5.0326x; 1.0187x over previous
import jax
import jax.numpy as jnp
from jax import lax
from jax.experimental import pallas as pl
from jax.experimental.pallas import tpu as pltpu
from jax.experimental.pallas import tpu_sc as plsc

EMBED_DIM = 32
BATCH = 16384
HIST = 200

NC = 2
NS = 16
NW = NC * NS

TOTAL = BATCH * HIST
PER_W = TOTAL // NW
CHUNK = 1600
NCHUNK = PER_W // CHUNK
NPAIR = NCHUNK // 2

assert PER_W * NW == TOTAL and NCHUNK * CHUNK == PER_W and NPAIR * 2 == NCHUNK


def _body(data_hbm, table_hbm, out_hbm,
          idx0, idx1, rows0, rows1, si0, si1, sg0, sg1, so0, so1):
    idx = [idx0, idx1]
    rows = [rows0, rows1]
    si = [si0, si1]
    sg = [sg0, sg1]
    so = [so0, so1]

    c = lax.axis_index("c")
    s = lax.axis_index("s")
    base = (s * NC + c) * PER_W

    def idx_copy(i, b):
        return pltpu.async_copy(
            data_hbm.at[pl.ds(base + i * CHUNK, CHUNK)], idx[b], si[b])

    def gather(b):
        return pltpu.async_copy(table_hbm.at[idx[b]], rows[b], sg[b])

    def store(i, b):
        return pltpu.async_copy(
            rows[b], out_hbm.at[pl.ds(base + i * CHUNK, CHUNK)], so[b])

    def wait_idx(b):
        pltpu.make_async_copy(data_hbm.at[pl.ds(0, CHUNK)], idx[b], si[b]).wait()

    def wait_gather(b):
        pltpu.make_async_copy(table_hbm.at[idx[b]], rows[b], sg[b]).wait()

    def wait_store(b):
        pltpu.make_async_copy(rows[b], out_hbm.at[pl.ds(0, CHUNK)], so[b]).wait()

    idx_copy(0, 0).wait()
    gather(0)
    idx_copy(1, 1)

    def pair_step(g, carry):
        for b in range(2):
            i = 2 * g + b
            bp = b ^ 1
            wait_gather(b)
            store(i, b)

            @pl.when(g < NPAIR - 1)
            def _():
                idx_copy(i + 2, b)

            if b == 0:
                @pl.when(g >= 1)
                def _():
                    wait_store(bp)
                wait_idx(bp)
                gather(bp)
            else:
                wait_store(bp)

                @pl.when(g < NPAIR - 1)
                def _():
                    wait_idx(bp)
                    gather(bp)
        return carry

    lax.fori_loop(0, NPAIR, pair_step, 0)
    wait_store(1)


@jax.jit
def kernel(data, ivectors):
    flat_idx = data.reshape(TOTAL)
    mesh = plsc.VectorSubcoreMesh(core_axis_name="c", subcore_axis_name="s")
    out = pl.kernel(
        _body,
        out_type=jax.ShapeDtypeStruct((TOTAL, EMBED_DIM), jnp.float32),
        mesh=mesh,
        scratch_types=[
            pltpu.VMEM((CHUNK,), jnp.int32),
            pltpu.VMEM((CHUNK,), jnp.int32),
            pltpu.VMEM((CHUNK, EMBED_DIM), jnp.float32),
            pltpu.VMEM((CHUNK, EMBED_DIM), jnp.float32),
            pltpu.SemaphoreType.DMA,
            pltpu.SemaphoreType.DMA,
            pltpu.SemaphoreType.DMA,
            pltpu.SemaphoreType.DMA,
            pltpu.SemaphoreType.DMA,
            pltpu.SemaphoreType.DMA,
        ],
        compiler_params=pltpu.CompilerParams(use_tc_tiling_on_sc=False),
    )(flat_idx, ivectors)
    return out.reshape(BATCH, HIST, EMBED_DIM)
